# baseline (speedup 1.0000x reference)
import jax
import jax.numpy as jnp
from jax.experimental import pallas as pl
from jax.experimental.pallas import tpu as pltpu

_B = 32
_H = 512
_W = 512

_BR = 256
_RG = 32
_RING = 5

_TC_BLOCKS = [(b, r0) for b in range(_B) for r0 in range(0, _H, _BR)]


def _body(m_hbm, lv_hbm, t_hbm, out_ref, mb, lvb, tb, sems):
    nb = len(_TC_BLOCKS)

    def start(g):
        b, r0 = _TC_BLOCKS[g]
        slot = g % _RING
        rows = pl.ds(r0, _BR)
        return (
            pltpu.async_copy(m_hbm.at[b, 0, rows], mb.at[slot], sems.at[slot]),
            pltpu.async_copy(lv_hbm.at[b, 0, rows], lvb.at[slot],
                             sems.at[slot]),
            pltpu.async_copy(t_hbm.at[b, rows], tb.at[slot], sems.at[slot]),
        )

    ahead = _RING - 1
    handles = [None] * nb
    for g in range(min(ahead, nb)):
        handles[g] = start(g)
    s = jnp.zeros((8, _W), jnp.float32)
    c = jnp.zeros((8, _W), jnp.float32)
    for g in range(nb):
        if g + ahead < nb:
            handles[g + ahead] = start(g + ahead)
        for h in handles[g]:
            h.wait()
        handles[g] = None
        slot = g % _RING
        for r in range(0, _BR, _RG):
            rows = pl.ds(r, _RG)
            m = mb[slot, rows, :]
            lv = lvb[slot, rows, :]
            t = tb[slot, rows, :]
            msk = t > 0.0
            d = t - m
            e = jnp.exp(-lv) * (d * d) + lv
            s = s + jnp.where(msk, e, 0.0).reshape(_RG // 8, 8, _W).sum(0)
            c = c + jnp.where(msk, 1.0, 0.0).reshape(_RG // 8, 8, _W).sum(0)
    out_ref[0, 0] = jnp.sum(s) / jnp.sum(c)


_loss_call = pl.pallas_call(
    _body,
    in_specs=[
        pl.BlockSpec(memory_space=pl.ANY),
        pl.BlockSpec(memory_space=pl.ANY),
        pl.BlockSpec(memory_space=pl.ANY),
    ],
    out_specs=pl.BlockSpec(memory_space=pltpu.MemorySpace.SMEM),
    out_shape=jax.ShapeDtypeStruct((1, 1), jnp.float32),
    scratch_shapes=[
        pltpu.VMEM((_RING, _BR, _W), jnp.float32),
        pltpu.VMEM((_RING, _BR, _W), jnp.float32),
        pltpu.VMEM((_RING, _BR, _W), jnp.float32),
        pltpu.SemaphoreType.DMA((_RING,)),
    ],
)


@jax.jit
def kernel(means, log_vars, targets):
    loss = _loss_call(means, log_vars, targets)
    return loss[0, 0]
